# baseline (speedup 1.0000x reference)
import jax
import jax.numpy as jnp
from jax import lax
from jax.experimental import pallas as pl
from jax.experimental.pallas import tpu as pltpu
from jax.experimental.pallas import tpu_sc as plsc

NC, NS, LANES = 2, 16, 16
NW = NC * NS
EMBED = 32
KB = 8
HPAD = 208


def _sc_body(ids_hbm, ctx_hbm, base_hbm, sup_hbm, out_hbm,
             idx_v, ctx_v, brows, srows, sem):
    nb, hist = ids_hbm.shape
    rows_per_w = nb // NW
    n_chunks = rows_per_w // KB
    wid = lax.axis_index("s") * NC + lax.axis_index("c")

    for rb in range(KB):
        idx_v[rb, pl.ds(192, LANES)] = jnp.zeros((LANES,), jnp.int32)

    def chunk_body(i, carry):
        b0 = wid * rows_per_w + i * KB
        for rb in range(KB):
            pltpu.sync_copy(ids_hbm.at[b0 + rb, :], idx_v.at[rb, pl.ds(0, hist)])
            pltpu.sync_copy(ctx_hbm.at[b0 + rb, :], ctx_v.at[rb, pl.ds(0, hist)])
        copies = []
        for rb in range(KB):
            copies.append(pltpu.async_copy(
                base_hbm.at[idx_v.at[rb, pl.ds(0, 128)]], brows.at[rb, pl.ds(0, 128)], sem))
            copies.append(pltpu.async_copy(
                sup_hbm.at[idx_v.at[rb, pl.ds(0, 128)]], srows.at[rb, pl.ds(0, 128)], sem))
            copies.append(pltpu.async_copy(
                base_hbm.at[idx_v.at[rb, pl.ds(128, 80)]], brows.at[rb, pl.ds(128, 80)], sem))
            copies.append(pltpu.async_copy(
                sup_hbm.at[idx_v.at[rb, pl.ds(128, 80)]], srows.at[rb, pl.ds(128, 80)], sem))
        for c in copies:
            c.wait()

        def row_block(rb, _):
            def blk_body(hb, __):
                k0 = hb * LANES
                cvec = ctx_v[rb, pl.ds(k0, LANES)]
                for j in range(LANES):
                    cb = jnp.full((LANES,), cvec[j])
                    for h in range(EMBED // LANES):
                        sl = (rb, k0 + j, pl.ds(h * LANES, LANES))
                        plsc.addupdate(brows.at[sl], cb * srows[sl])
                return __
            return lax.fori_loop(0, HPAD // LANES, blk_body, _)

        lax.fori_loop(0, KB, row_block, 0)
        for rb in range(KB):
            pltpu.sync_copy(brows.at[rb, pl.ds(0, hist)], out_hbm.at[b0 + rb])
        return carry

    lax.fori_loop(0, n_chunks, chunk_body, 0)


def kernel(input_ids, context_vector, base_table, superposed_table):
    b, h = input_ids.shape
    mesh = plsc.VectorSubcoreMesh(core_axis_name="c", subcore_axis_name="s",
                                  num_cores=NC, num_subcores=NS)
    out = pl.kernel(
        _sc_body,
        out_type=jax.ShapeDtypeStruct((b, h, EMBED), jnp.float32),
        mesh=mesh,
        scratch_types=[
            pltpu.VMEM((KB, HPAD), jnp.int32),
            pltpu.VMEM((KB, HPAD), jnp.float32),
            pltpu.VMEM((KB, HPAD, EMBED), jnp.float32),
            pltpu.VMEM((KB, HPAD, EMBED), jnp.float32),
            pltpu.SemaphoreType.DMA,
        ],
        compiler_params=pltpu.CompilerParams(use_tc_tiling_on_sc=False),
    )(input_ids.astype(jnp.int32), context_vector, base_table, superposed_table)
    return out
